# Initial kernel scaffold; baseline (speedup 1.0000x reference)
import jax
import jax.numpy as jnp
from jax.experimental import pallas as pl


def kernel(x, edge_index, edge_attr, batch, Wf1, bf1, Ws1, bs1, Wf2, bf2, Ws2, bs2, Wfc, bfc):
    raise NotImplementedError("write your pallas kernel here")


# speedup vs baseline: 1.4202x; 1.4202x over previous
import functools

import jax
import jax.numpy as jnp
from jax import lax
from jax.experimental import pallas as pl
from jax.experimental.pallas import tpu as pltpu
from jax.experimental.pallas import tpu_sc as plsc

NN = 10000
NE = 160000
F = 256
DE = 16
NG = 64

NODE_BLK = 1000
EDGE_BLK = 1000
N_TILES = 16
KG = 80
KS = 128
EPT = NE // N_TILES


def _idx_body(ei_ref, out_ref):
    s = ei_ref[0:1, :]
    d = ei_ref[1:2, :]
    out_ref[0:1, :] = d * 4
    out_ref[1:2, :] = d * 4 + 1
    out_ref[2:3, :] = s * 4 + 2
    out_ref[3:4, :] = s * 4 + 3
    out_ref[4:5, :] = d
    out_ref[5:6, :] = d
    out_ref[6:7, :] = d
    out_ref[7:8, :] = d


def _build_idx(ei):
    return pl.pallas_call(
        _idx_body,
        out_shape=jax.ShapeDtypeStruct((8, NE), jnp.int32),
    )(ei)


def _tab1_body(x_ref, w_ref, p_ref):
    p_ref[...] = jnp.dot(x_ref[...], w_ref[...],
                         preferred_element_type=jnp.float32)


def _tables1(x, w4):
    return pl.pallas_call(
        _tab1_body,
        grid=(NN // NODE_BLK,),
        in_specs=[
            pl.BlockSpec((NODE_BLK, F), lambda i: (i, 0)),
            pl.BlockSpec((F, 4 * F), lambda i: (0, 0)),
        ],
        out_specs=pl.BlockSpec((NODE_BLK, 4 * F), lambda i: (i, 0)),
        out_shape=jax.ShapeDtypeStruct((NN, 4 * F), jnp.float32),
    )(x, w4)


def _tab2_body(x_ref, agg_ref, w_ref, h_ref, p_ref):
    h = x_ref[...] + jnp.concatenate([agg_ref[0], agg_ref[1]], axis=-1)
    h_ref[...] = h
    p_ref[...] = jnp.dot(h, w_ref[...], preferred_element_type=jnp.float32)


def _tables2(x, agg, w4):
    return pl.pallas_call(
        _tab2_body,
        grid=(NN // NODE_BLK,),
        in_specs=[
            pl.BlockSpec((NODE_BLK, F), lambda i: (i, 0)),
            pl.BlockSpec((2, NODE_BLK, F // 2), lambda i: (0, i, 0)),
            pl.BlockSpec((F, 4 * F), lambda i: (0, 0)),
        ],
        out_specs=[
            pl.BlockSpec((NODE_BLK, F), lambda i: (i, 0)),
            pl.BlockSpec((NODE_BLK, 4 * F), lambda i: (i, 0)),
        ],
        out_shape=[
            jax.ShapeDtypeStruct((NN, F), jnp.float32),
            jax.ShapeDtypeStruct((NN, 4 * F), jnp.float32),
        ],
    )(x, agg, w4)


def _edge_body(g_ref, ea_ref, we_ref, b_ref, m_ref):
    e = jnp.dot(ea_ref[...], we_ref[...], preferred_element_type=jnp.float32)
    zf = g_ref[0] + e[:, :F] + b_ref[0:1, :F]
    zs = g_ref[1] + e[:, F:] + b_ref[0:1, F:]
    gate = jax.nn.sigmoid(zf)
    sp = jnp.maximum(zs, 0.0) + jnp.log1p(jnp.exp(-jnp.abs(zs)))
    m = gate * sp
    m_ref[0] = m[:, : F // 2]
    m_ref[1] = m[:, F // 2:]


def _edge_mlp(g, ea, we, b):
    return pl.pallas_call(
        _edge_body,
        grid=(NE // EDGE_BLK,),
        in_specs=[
            pl.BlockSpec((2, EDGE_BLK, F), lambda i: (0, i, 0)),
            pl.BlockSpec((EDGE_BLK, DE), lambda i: (i, 0)),
            pl.BlockSpec((DE, 2 * F), lambda i: (0, 0)),
            pl.BlockSpec((8, 2 * F), lambda i: (0, 0)),
        ],
        out_specs=pl.BlockSpec((2, EDGE_BLK, F // 2), lambda i: (0, i, 0)),
        out_shape=jax.ShapeDtypeStruct((2, NE, F // 2), jnp.float32),
    )(g, ea, we, b)


def _final_body(h_ref, agg_ref, batch_ref, wfc_ref, bfc_ref, out_ref,
                pooled, cnt):
    i = pl.program_id(0)

    @pl.when(i == 0)
    def _():
        pooled[...] = jnp.zeros_like(pooled)
        cnt[...] = jnp.zeros_like(cnt)

    h2 = h_ref[...] + jnp.concatenate([agg_ref[0], agg_ref[1]], axis=-1)
    b = batch_ref[0, 0, :]
    ids = lax.broadcasted_iota(jnp.int32, (NG, NODE_BLK), 0)
    mask_t = (ids == b[None, :]).astype(jnp.float32)
    pooled[...] += jnp.dot(mask_t, h2, preferred_element_type=jnp.float32)
    cnt[...] += jnp.broadcast_to(
        jnp.sum(mask_t, axis=1, keepdims=True), (NG, 128))

    @pl.when(i == pl.num_programs(0) - 1)
    def _():
        out_ref[...] = (jnp.dot(pooled[...], wfc_ref[...],
                                preferred_element_type=jnp.float32)
                        + cnt[:, 0:1] * bfc_ref[0, 0])


def _final(h1, agg2, batch3, wfc, bfc11):
    return pl.pallas_call(
        _final_body,
        grid=(NN // NODE_BLK,),
        in_specs=[
            pl.BlockSpec((NODE_BLK, F), lambda i: (i, 0)),
            pl.BlockSpec((2, NODE_BLK, F // 2), lambda i: (0, i, 0)),
            pl.BlockSpec((1, 1, NODE_BLK), lambda i: (i, 0, 0)),
            pl.BlockSpec((F, 1), lambda i: (0, 0)),
            pl.BlockSpec((1, 1), lambda i: (0, 0),
                         memory_space=pltpu.SMEM),
        ],
        out_specs=pl.BlockSpec((NG, 1), lambda i: (0, 0)),
        out_shape=jax.ShapeDtypeStruct((NG, 1), jnp.float32),
        scratch_shapes=[
            pltpu.VMEM((NG, F), jnp.float32),
            pltpu.VMEM((NG, 128), jnp.float32),
        ],
    )(h1, agg2, batch3, wfc, bfc11)


def _mesh():
    return plsc.VectorSubcoreMesh(core_axis_name="c", subcore_axis_name="s",
                                  num_cores=2, num_subcores=N_TILES)


def _sc_gather(p40, idx8):
    @functools.partial(
        pl.kernel,
        out_type=jax.ShapeDtypeStruct((2, NE, F), jnp.float32),
        mesh=_mesh(),
        scratch_types=[
            pltpu.VMEM((KG,), jnp.int32),
            pltpu.VMEM((KG,), jnp.int32),
            pltpu.VMEM((KG, F), jnp.float32),
            pltpu.VMEM((KG, F), jnp.float32),
            pltpu.SemaphoreType.DMA,
            pltpu.SemaphoreType.DMA,
        ],
    )
    def k(p_hbm, i8_hbm, g_hbm, idxd, idxs, bufd, bufs, semd, sems):
        c = lax.axis_index("c")
        s = lax.axis_index("s")
        base = s * EPT

        @pl.loop(0, EPT // KG)
        def _chunk(kk):
            e0 = base + kk * KG
            pltpu.sync_copy(i8_hbm.at[pl.ds(c * NE + e0, KG)], idxd)
            pltpu.sync_copy(i8_hbm.at[pl.ds((2 + c) * NE + e0, KG)], idxs)
            cpd = pltpu.async_copy(p_hbm.at[idxd], bufd, semd)
            cps = pltpu.async_copy(p_hbm.at[idxs], bufs, sems)
            cpd.wait()
            cps.wait()

            @pl.loop(0, KG)
            def _row(i):
                for j in range(F // 16):
                    sl = pl.ds(j * 16, 16)
                    bufd[i, sl] = bufd[i, sl] + bufs[i, sl]

            pltpu.sync_copy(bufd, g_hbm.at[c, pl.ds(e0, KG), :])

    return k(p40, idx8)


def _sc_scatter(m2, idx8):
    n_chunks = NE // KS
    zrows = 80
    n_zchunks = NN // zrows

    @functools.partial(
        pl.kernel,
        out_type=jax.ShapeDtypeStruct((2, NN, F // 2), jnp.float32),
        mesh=_mesh(),
        scratch_types=[
            pltpu.VMEM((KS,), jnp.int32),
            pltpu.VMEM((KS, F // 2), jnp.float32),
            pltpu.VMEM((80, F // 2), jnp.float32),
            pltpu.VMEM_SHARED((NN, F // 2), jnp.float32),
        ],
    )
    def k(m_hbm, i8_hbm, agg_hbm, idxv, mbuf, zbuf, acc):
        c = lax.axis_index("c")
        s = lax.axis_index("s")

        @pl.loop(0, zrows)
        def _zrow(i):
            for j in range(F // 2 // 16):
                zbuf[i, pl.ds(j * 16, 16)] = jnp.zeros((16,), jnp.float32)

        @pl.loop(0, (n_zchunks + N_TILES - 1) // N_TILES)
        def _zcp(t):
            ch = t * N_TILES + s

            @pl.when(ch < n_zchunks)
            def _():
                pltpu.sync_copy(zbuf, acc.at[pl.ds(ch * zrows, zrows), :])

        plsc.subcore_barrier()

        @pl.loop(0, n_chunks // N_TILES + 1)
        def _chunk(kk):
            ch = kk * N_TILES + s

            @pl.when(ch < n_chunks)
            def _():
                e0 = ch * KS
                pltpu.sync_copy(i8_hbm.at[pl.ds(4 * NE + e0, KS)], idxv)
                pltpu.sync_copy(m_hbm.at[c, pl.ds(e0, KS), :], mbuf)
                pltpu.sync_copy(mbuf, acc.at[idxv], add=True)

        plsc.subcore_barrier()

        @pl.loop(0, (n_zchunks + N_TILES - 1) // N_TILES)
        def _dump(t):
            ch = t * N_TILES + s

            @pl.when(ch < n_zchunks)
            def _():
                r0 = ch * zrows
                pltpu.sync_copy(acc.at[pl.ds(r0, zrows), :],
                                agg_hbm.at[c, pl.ds(r0, zrows), :])

    return k(m2, idx8)


def kernel(x, edge_index, edge_attr, batch,
           Wf1, bf1, Ws1, bs1, Wf2, bf2, Ws2, bs2, Wfc, bfc):
    ei = edge_index.astype(jnp.int32)
    idx8 = _build_idx(ei)

    w4_1 = jnp.concatenate(
        [Wf1[:F], Ws1[:F], Wf1[F:2 * F], Ws1[F:2 * F]], axis=1)
    w4_2 = jnp.concatenate(
        [Wf2[:F], Ws2[:F], Wf2[F:2 * F], Ws2[F:2 * F]], axis=1)
    we1 = jnp.concatenate([Wf1[2 * F:], Ws1[2 * F:]], axis=1)
    we2 = jnp.concatenate([Wf2[2 * F:], Ws2[2 * F:]], axis=1)
    b1 = jnp.broadcast_to(jnp.concatenate([bf1, bs1])[None, :], (8, 2 * F))
    b2 = jnp.broadcast_to(jnp.concatenate([bf2, bs2])[None, :], (8, 2 * F))

    p1 = _tables1(x, w4_1)
    idxf = idx8.reshape(8 * NE)
    g1 = _sc_gather(p1.reshape(4 * NN, F), idxf)
    m1 = _edge_mlp(g1, edge_attr, we1, b1)
    agg1 = _sc_scatter(m1, idxf)

    h1, p2 = _tables2(x, agg1, w4_2)
    g2 = _sc_gather(p2.reshape(4 * NN, F), idxf)
    m2 = _edge_mlp(g2, edge_attr, we2, b2)
    agg2 = _sc_scatter(m2, idxf)

    batch3 = batch.astype(jnp.int32).reshape(NN // NODE_BLK, 1, NODE_BLK)
    return _final(h1, agg2, batch3, Wfc, jnp.reshape(bfc, (1, 1)))


# speedup vs baseline: 1.9502x; 1.3731x over previous
import functools

import jax
import jax.numpy as jnp
from jax import lax
from jax.experimental import pallas as pl
from jax.experimental.pallas import tpu as pltpu
from jax.experimental.pallas import tpu_sc as plsc

NN = 10000
NE = 160000
F = 256
DE = 16
NG = 64

NODE_BLK = 1000
EDGE_BLK = 1000
N_TILES = 16
KG = 80
KS = 128
EPT = NE // N_TILES


def _idx_body(ei_ref, out_ref):
    s = ei_ref[0:1, :]
    d = ei_ref[1:2, :]
    out_ref[0:1, :] = d * 4
    out_ref[1:2, :] = d * 4 + 1
    out_ref[2:3, :] = s * 4 + 2
    out_ref[3:4, :] = s * 4 + 3
    out_ref[4:5, :] = d
    out_ref[5:6, :] = d
    out_ref[6:7, :] = d
    out_ref[7:8, :] = d


def _build_idx(ei):
    return pl.pallas_call(
        _idx_body,
        out_shape=jax.ShapeDtypeStruct((8, NE), jnp.int32),
    )(ei)


def _tab1_body(x_ref, w_ref, p_ref):
    p_ref[...] = jnp.dot(x_ref[...], w_ref[...],
                         preferred_element_type=jnp.float32)


def _tables1(x, w4):
    return pl.pallas_call(
        _tab1_body,
        grid=(NN // NODE_BLK,),
        in_specs=[
            pl.BlockSpec((NODE_BLK, F), lambda i: (i, 0)),
            pl.BlockSpec((F, 4 * F), lambda i: (0, 0)),
        ],
        out_specs=pl.BlockSpec((NODE_BLK, 4 * F), lambda i: (i, 0)),
        out_shape=jax.ShapeDtypeStruct((NN, 4 * F), jnp.float32),
    )(x, w4)


def _tab2_body(x_ref, agg_ref, w_ref, h_ref, p_ref):
    h = x_ref[...] + jnp.concatenate([agg_ref[0], agg_ref[1]], axis=-1)
    h_ref[...] = h
    p_ref[...] = jnp.dot(h, w_ref[...], preferred_element_type=jnp.float32)


def _tables2(x, agg, w4):
    return pl.pallas_call(
        _tab2_body,
        grid=(NN // NODE_BLK,),
        in_specs=[
            pl.BlockSpec((NODE_BLK, F), lambda i: (i, 0)),
            pl.BlockSpec((2, NODE_BLK, F // 2), lambda i: (0, i, 0)),
            pl.BlockSpec((F, 4 * F), lambda i: (0, 0)),
        ],
        out_specs=[
            pl.BlockSpec((NODE_BLK, F), lambda i: (i, 0)),
            pl.BlockSpec((NODE_BLK, 4 * F), lambda i: (i, 0)),
        ],
        out_shape=[
            jax.ShapeDtypeStruct((NN, F), jnp.float32),
            jax.ShapeDtypeStruct((NN, 4 * F), jnp.float32),
        ],
    )(x, agg, w4)


def _edge_body(g_ref, ea_ref, we_ref, b_ref, m_ref):
    e = jnp.dot(ea_ref[...], we_ref[...], preferred_element_type=jnp.float32)
    zf = g_ref[0] + e[:, :F] + b_ref[0:1, :F]
    zs = g_ref[1] + e[:, F:] + b_ref[0:1, F:]
    gate = jax.nn.sigmoid(zf)
    sp = jnp.maximum(zs, 0.0) + jnp.log1p(jnp.exp(-jnp.abs(zs)))
    m = gate * sp
    m_ref[0] = m[:, : F // 2]
    m_ref[1] = m[:, F // 2:]


def _edge_mlp(g, ea, we, b):
    return pl.pallas_call(
        _edge_body,
        grid=(NE // EDGE_BLK,),
        in_specs=[
            pl.BlockSpec((2, EDGE_BLK, F), lambda i: (0, i, 0)),
            pl.BlockSpec((EDGE_BLK, DE), lambda i: (i, 0)),
            pl.BlockSpec((DE, 2 * F), lambda i: (0, 0)),
            pl.BlockSpec((8, 2 * F), lambda i: (0, 0)),
        ],
        out_specs=pl.BlockSpec((2, EDGE_BLK, F // 2), lambda i: (0, i, 0)),
        out_shape=jax.ShapeDtypeStruct((2, NE, F // 2), jnp.float32),
    )(g, ea, we, b)


def _final_body(h_ref, agg_ref, batch_ref, wfc_ref, bfc_ref, out_ref,
                pooled, cnt):
    i = pl.program_id(0)

    @pl.when(i == 0)
    def _():
        pooled[...] = jnp.zeros_like(pooled)
        cnt[...] = jnp.zeros_like(cnt)

    h2 = h_ref[...] + jnp.concatenate([agg_ref[0], agg_ref[1]], axis=-1)
    b = batch_ref[0, 0, :]
    ids = lax.broadcasted_iota(jnp.int32, (NG, NODE_BLK), 0)
    mask_t = (ids == b[None, :]).astype(jnp.float32)
    pooled[...] += jnp.dot(mask_t, h2, preferred_element_type=jnp.float32)
    cnt[...] += jnp.broadcast_to(
        jnp.sum(mask_t, axis=1, keepdims=True), (NG, 128))

    @pl.when(i == pl.num_programs(0) - 1)
    def _():
        out_ref[...] = (jnp.dot(pooled[...], wfc_ref[...],
                                preferred_element_type=jnp.float32)
                        + cnt[:, 0:1] * bfc_ref[0, 0])


def _final(h1, agg2, batch3, wfc, bfc11):
    return pl.pallas_call(
        _final_body,
        grid=(NN // NODE_BLK,),
        in_specs=[
            pl.BlockSpec((NODE_BLK, F), lambda i: (i, 0)),
            pl.BlockSpec((2, NODE_BLK, F // 2), lambda i: (0, i, 0)),
            pl.BlockSpec((1, 1, NODE_BLK), lambda i: (i, 0, 0)),
            pl.BlockSpec((F, 1), lambda i: (0, 0)),
            pl.BlockSpec((1, 1), lambda i: (0, 0),
                         memory_space=pltpu.SMEM),
        ],
        out_specs=pl.BlockSpec((NG, 1), lambda i: (0, 0)),
        out_shape=jax.ShapeDtypeStruct((NG, 1), jnp.float32),
        scratch_shapes=[
            pltpu.VMEM((NG, F), jnp.float32),
            pltpu.VMEM((NG, 128), jnp.float32),
        ],
    )(h1, agg2, batch3, wfc, bfc11)


def _mesh():
    return plsc.VectorSubcoreMesh(core_axis_name="c", subcore_axis_name="s",
                                  num_cores=2, num_subcores=N_TILES)


def _sc_gather(p40, idx8):
    nch = EPT // KG

    @functools.partial(
        pl.kernel,
        out_type=jax.ShapeDtypeStruct((2, NE, F), jnp.float32),
        mesh=_mesh(),
        scratch_types=[
            pltpu.VMEM((EPT,), jnp.int32),
            pltpu.VMEM((EPT,), jnp.int32),
            pltpu.VMEM((KG, F), jnp.float32),
            pltpu.VMEM((KG, F), jnp.float32),
            pltpu.VMEM((KG, F), jnp.float32),
            pltpu.VMEM((KG, F), jnp.float32),
            pltpu.SemaphoreType.DMA,
            pltpu.SemaphoreType.DMA,
            pltpu.SemaphoreType.DMA,
            pltpu.SemaphoreType.DMA,
        ],
    )
    def k(p_hbm, i8_hbm, g_hbm, idxd, idxs,
          bda, bsa, bdb, bsb, sda, ssa, sdb, ssb):
        c = lax.axis_index("c")
        s = lax.axis_index("s")
        base = s * EPT
        pltpu.sync_copy(i8_hbm.at[pl.ds(c * NE + base, EPT)], idxd)
        pltpu.sync_copy(i8_hbm.at[pl.ds((2 + c) * NE + base, EPT)], idxs)

        def start(a, bufd, bufs, semd, sems):
            sl = pl.ds(a * KG, KG)
            pltpu.async_copy(p_hbm.at[idxd.at[sl]], bufd, semd)
            pltpu.async_copy(p_hbm.at[idxs.at[sl]], bufs, sems)

        def finish(a, bufd, bufs, semd, sems):
            sl = pl.ds(a * KG, KG)
            pltpu.make_async_copy(p_hbm.at[idxd.at[sl]], bufd, semd).wait()
            pltpu.make_async_copy(p_hbm.at[idxs.at[sl]], bufs, sems).wait()

            @pl.loop(0, KG)
            def _row(i):
                for j in range(F // 16):
                    fsl = pl.ds(j * 16, 16)
                    bufd[i, fsl] = bufd[i, fsl] + bufs[i, fsl]

            pltpu.sync_copy(bufd, g_hbm.at[c, pl.ds(base + a * KG, KG), :])

        start(0, bda, bsa, sda, ssa)

        @pl.loop(0, nch // 2)
        def _pair(p):
            a = 2 * p
            start(a + 1, bdb, bsb, sdb, ssb)
            finish(a, bda, bsa, sda, ssa)

            @pl.when(a + 2 < nch)
            def _():
                start(a + 2, bda, bsa, sda, ssa)

            finish(a + 1, bdb, bsb, sdb, ssb)

        finish(nch - 1, bda, bsa, sda, ssa)

    return k(p40, idx8)


def _sc_scatter(m2, idx8):
    n_chunks = NE // KS
    zrows = 80
    n_zchunks = NN // zrows

    @functools.partial(
        pl.kernel,
        out_type=jax.ShapeDtypeStruct((2, NN, F // 2), jnp.float32),
        mesh=_mesh(),
        scratch_types=[
            pltpu.VMEM((KS,), jnp.int32),
            pltpu.VMEM((KS, F // 2), jnp.float32),
            pltpu.VMEM((80, F // 2), jnp.float32),
            pltpu.VMEM_SHARED((NN, F // 2), jnp.float32),
        ],
    )
    def k(m_hbm, i8_hbm, agg_hbm, idxv, mbuf, zbuf, acc):
        c = lax.axis_index("c")
        s = lax.axis_index("s")

        @pl.loop(0, zrows)
        def _zrow(i):
            for j in range(F // 2 // 16):
                zbuf[i, pl.ds(j * 16, 16)] = jnp.zeros((16,), jnp.float32)

        @pl.loop(0, (n_zchunks + N_TILES - 1) // N_TILES)
        def _zcp(t):
            ch = t * N_TILES + s

            @pl.when(ch < n_zchunks)
            def _():
                pltpu.sync_copy(zbuf, acc.at[pl.ds(ch * zrows, zrows), :])

        plsc.subcore_barrier()

        @pl.loop(0, n_chunks // N_TILES + 1)
        def _chunk(kk):
            ch = kk * N_TILES + s

            @pl.when(ch < n_chunks)
            def _():
                e0 = ch * KS
                pltpu.sync_copy(i8_hbm.at[pl.ds(4 * NE + e0, KS)], idxv)
                pltpu.sync_copy(m_hbm.at[c, pl.ds(e0, KS), :], mbuf)
                pltpu.sync_copy(mbuf, acc.at[idxv], add=True)

        plsc.subcore_barrier()

        @pl.loop(0, (n_zchunks + N_TILES - 1) // N_TILES)
        def _dump(t):
            ch = t * N_TILES + s

            @pl.when(ch < n_zchunks)
            def _():
                r0 = ch * zrows
                pltpu.sync_copy(acc.at[pl.ds(r0, zrows), :],
                                agg_hbm.at[c, pl.ds(r0, zrows), :])

    return k(m2, idx8)


def kernel(x, edge_index, edge_attr, batch,
           Wf1, bf1, Ws1, bs1, Wf2, bf2, Ws2, bs2, Wfc, bfc):
    ei = edge_index.astype(jnp.int32)
    idx8 = _build_idx(ei)

    w4_1 = jnp.concatenate(
        [Wf1[:F], Ws1[:F], Wf1[F:2 * F], Ws1[F:2 * F]], axis=1)
    w4_2 = jnp.concatenate(
        [Wf2[:F], Ws2[:F], Wf2[F:2 * F], Ws2[F:2 * F]], axis=1)
    we1 = jnp.concatenate([Wf1[2 * F:], Ws1[2 * F:]], axis=1)
    we2 = jnp.concatenate([Wf2[2 * F:], Ws2[2 * F:]], axis=1)
    b1 = jnp.broadcast_to(jnp.concatenate([bf1, bs1])[None, :], (8, 2 * F))
    b2 = jnp.broadcast_to(jnp.concatenate([bf2, bs2])[None, :], (8, 2 * F))

    p1 = _tables1(x, w4_1)
    idxf = idx8.reshape(8 * NE)
    g1 = _sc_gather(p1.reshape(4 * NN, F), idxf)
    m1 = _edge_mlp(g1, edge_attr, we1, b1)
    agg1 = _sc_scatter(m1, idxf)

    h1, p2 = _tables2(x, agg1, w4_2)
    g2 = _sc_gather(p2.reshape(4 * NN, F), idxf)
    m2 = _edge_mlp(g2, edge_attr, we2, b2)
    agg2 = _sc_scatter(m2, idxf)

    batch3 = batch.astype(jnp.int32).reshape(NN // NODE_BLK, 1, NODE_BLK)
    return _final(h1, agg2, batch3, Wfc, jnp.reshape(bfc, (1, 1)))


# speedup vs baseline: 2.1688x; 1.1121x over previous
import functools

import jax
import jax.numpy as jnp
from jax import lax
from jax.experimental import pallas as pl
from jax.experimental.pallas import tpu as pltpu
from jax.experimental.pallas import tpu_sc as plsc

NN = 10000
NE = 160000
F = 256
DE = 16
NG = 64

NODE_BLK = 1000
EDGE_BLK = 1000
N_TILES = 16
KG = 80
KS = 80
EPT = NE // N_TILES


def _idx_body(ei_ref, out_ref):
    s = ei_ref[0:1, :]
    d = ei_ref[1:2, :]
    out_ref[0:1, :] = d * 4
    out_ref[1:2, :] = d * 4 + 1
    out_ref[2:3, :] = s * 4 + 2
    out_ref[3:4, :] = s * 4 + 3
    out_ref[4:5, :] = d
    out_ref[5:6, :] = d
    out_ref[6:7, :] = d
    out_ref[7:8, :] = d


def _build_idx(ei):
    return pl.pallas_call(
        _idx_body,
        out_shape=jax.ShapeDtypeStruct((8, NE), jnp.int32),
    )(ei)


def _tab1_body(x_ref, w_ref, p_ref):
    p_ref[...] = jnp.dot(x_ref[...], w_ref[...],
                         preferred_element_type=jnp.float32)


def _tables1(x, w4):
    return pl.pallas_call(
        _tab1_body,
        grid=(NN // NODE_BLK,),
        in_specs=[
            pl.BlockSpec((NODE_BLK, F), lambda i: (i, 0)),
            pl.BlockSpec((F, 4 * F), lambda i: (0, 0)),
        ],
        out_specs=pl.BlockSpec((NODE_BLK, 4 * F), lambda i: (i, 0)),
        out_shape=jax.ShapeDtypeStruct((NN, 4 * F), jnp.float32),
    )(x, w4)


def _tab2_body(x_ref, agg_ref, w_ref, h_ref, p_ref):
    h = x_ref[...] + jnp.concatenate([agg_ref[0], agg_ref[1]], axis=-1)
    h_ref[...] = h
    p_ref[...] = jnp.dot(h, w_ref[...], preferred_element_type=jnp.float32)


def _tables2(x, agg, w4):
    return pl.pallas_call(
        _tab2_body,
        grid=(NN // NODE_BLK,),
        in_specs=[
            pl.BlockSpec((NODE_BLK, F), lambda i: (i, 0)),
            pl.BlockSpec((2, NODE_BLK, F // 2), lambda i: (0, i, 0)),
            pl.BlockSpec((F, 4 * F), lambda i: (0, 0)),
        ],
        out_specs=[
            pl.BlockSpec((NODE_BLK, F), lambda i: (i, 0)),
            pl.BlockSpec((NODE_BLK, 4 * F), lambda i: (i, 0)),
        ],
        out_shape=[
            jax.ShapeDtypeStruct((NN, F), jnp.float32),
            jax.ShapeDtypeStruct((NN, 4 * F), jnp.float32),
        ],
    )(x, agg, w4)


def _edge_body(g_ref, ea_ref, we_ref, b_ref, m_ref):
    e = jnp.dot(ea_ref[...], we_ref[...], preferred_element_type=jnp.float32)
    zf = g_ref[0] + e[:, :F] + b_ref[0:1, :F]
    zs = g_ref[1] + e[:, F:] + b_ref[0:1, F:]
    gate = jax.nn.sigmoid(zf)
    sp = jnp.maximum(zs, 0.0) + jnp.log1p(jnp.exp(-jnp.abs(zs)))
    m = gate * sp
    m_ref[0] = m[:, : F // 2]
    m_ref[1] = m[:, F // 2:]


def _edge_mlp(g, ea, we, b):
    return pl.pallas_call(
        _edge_body,
        grid=(NE // EDGE_BLK,),
        in_specs=[
            pl.BlockSpec((2, EDGE_BLK, F), lambda i: (0, i, 0)),
            pl.BlockSpec((EDGE_BLK, DE), lambda i: (i, 0)),
            pl.BlockSpec((DE, 2 * F), lambda i: (0, 0)),
            pl.BlockSpec((8, 2 * F), lambda i: (0, 0)),
        ],
        out_specs=pl.BlockSpec((2, EDGE_BLK, F // 2), lambda i: (0, i, 0)),
        out_shape=jax.ShapeDtypeStruct((2, NE, F // 2), jnp.float32),
    )(g, ea, we, b)


def _final_body(h_ref, agg_ref, batch_ref, wfc_ref, bfc_ref, out_ref,
                pooled, cnt):
    i = pl.program_id(0)

    @pl.when(i == 0)
    def _():
        pooled[...] = jnp.zeros_like(pooled)
        cnt[...] = jnp.zeros_like(cnt)

    h2 = h_ref[...] + jnp.concatenate([agg_ref[0], agg_ref[1]], axis=-1)
    b = batch_ref[0, 0, :]
    ids = lax.broadcasted_iota(jnp.int32, (NG, NODE_BLK), 0)
    mask_t = (ids == b[None, :]).astype(jnp.float32)
    pooled[...] += jnp.dot(mask_t, h2, preferred_element_type=jnp.float32)
    cnt[...] += jnp.broadcast_to(
        jnp.sum(mask_t, axis=1, keepdims=True), (NG, 128))

    @pl.when(i == pl.num_programs(0) - 1)
    def _():
        out_ref[...] = (jnp.dot(pooled[...], wfc_ref[...],
                                preferred_element_type=jnp.float32)
                        + cnt[:, 0:1] * bfc_ref[0, 0])


def _final(h1, agg2, batch3, wfc, bfc11):
    return pl.pallas_call(
        _final_body,
        grid=(NN // NODE_BLK,),
        in_specs=[
            pl.BlockSpec((NODE_BLK, F), lambda i: (i, 0)),
            pl.BlockSpec((2, NODE_BLK, F // 2), lambda i: (0, i, 0)),
            pl.BlockSpec((1, 1, NODE_BLK), lambda i: (i, 0, 0)),
            pl.BlockSpec((F, 1), lambda i: (0, 0)),
            pl.BlockSpec((1, 1), lambda i: (0, 0),
                         memory_space=pltpu.SMEM),
        ],
        out_specs=pl.BlockSpec((NG, 1), lambda i: (0, 0)),
        out_shape=jax.ShapeDtypeStruct((NG, 1), jnp.float32),
        scratch_shapes=[
            pltpu.VMEM((NG, F), jnp.float32),
            pltpu.VMEM((NG, 128), jnp.float32),
        ],
    )(h1, agg2, batch3, wfc, bfc11)


def _mesh():
    return plsc.VectorSubcoreMesh(core_axis_name="c", subcore_axis_name="s",
                                  num_cores=2, num_subcores=N_TILES)


def _sc_gather(p40, idx8):
    nch = EPT // KG

    @functools.partial(
        pl.kernel,
        out_type=jax.ShapeDtypeStruct((2, NE, F), jnp.float32),
        mesh=_mesh(),
        scratch_types=[
            pltpu.VMEM((EPT,), jnp.int32),
            pltpu.VMEM((EPT,), jnp.int32),
            pltpu.VMEM((KG, F), jnp.float32),
            pltpu.VMEM((KG, F), jnp.float32),
            pltpu.VMEM((KG, F), jnp.float32),
            pltpu.VMEM((KG, F), jnp.float32),
            pltpu.SemaphoreType.DMA,
            pltpu.SemaphoreType.DMA,
            pltpu.SemaphoreType.DMA,
            pltpu.SemaphoreType.DMA,
        ],
    )
    def k(p_hbm, i8_hbm, g_hbm, idxd, idxs,
          bda, bsa, bdb, bsb, sda, ssa, sdb, ssb):
        c = lax.axis_index("c")
        s = lax.axis_index("s")
        base = s * EPT
        pltpu.sync_copy(i8_hbm.at[pl.ds(c * NE + base, EPT)], idxd)
        pltpu.sync_copy(i8_hbm.at[pl.ds((2 + c) * NE + base, EPT)], idxs)

        def start(a, bufd, bufs, semd, sems):
            sl = pl.ds(a * KG, KG)
            pltpu.async_copy(p_hbm.at[idxd.at[sl]], bufd, semd)
            pltpu.async_copy(p_hbm.at[idxs.at[sl]], bufs, sems)

        def finish(a, bufd, bufs, semd, sems):
            sl = pl.ds(a * KG, KG)
            pltpu.make_async_copy(p_hbm.at[idxd.at[sl]], bufd, semd).wait()
            pltpu.make_async_copy(p_hbm.at[idxs.at[sl]], bufs, sems).wait()

            @pl.loop(0, KG)
            def _row(i):
                for j in range(F // 16):
                    fsl = pl.ds(j * 16, 16)
                    bufd[i, fsl] = bufd[i, fsl] + bufs[i, fsl]

            pltpu.sync_copy(bufd, g_hbm.at[c, pl.ds(base + a * KG, KG), :])

        start(0, bda, bsa, sda, ssa)

        @pl.loop(0, nch // 2)
        def _pair(p):
            a = 2 * p
            start(a + 1, bdb, bsb, sdb, ssb)
            finish(a, bda, bsa, sda, ssa)

            @pl.when(a + 2 < nch)
            def _():
                start(a + 2, bda, bsa, sda, ssa)

            finish(a + 1, bdb, bsb, sdb, ssb)

        finish(nch - 1, bda, bsa, sda, ssa)

    return k(p40, idx8)


def _sc_scatter(m2, idx8):
    nch = EPT // KS
    zrows = 80
    n_zchunks = NN // zrows

    @functools.partial(
        pl.kernel,
        out_type=jax.ShapeDtypeStruct((2, NN, F // 2), jnp.float32),
        mesh=_mesh(),
        scratch_types=[
            pltpu.VMEM((KS,), jnp.int32),
            pltpu.VMEM((KS,), jnp.int32),
            pltpu.VMEM((KS, F // 2), jnp.float32),
            pltpu.VMEM((KS, F // 2), jnp.float32),
            pltpu.VMEM((zrows, F // 2), jnp.float32),
            pltpu.VMEM_SHARED((NN, F // 2), jnp.float32),
            pltpu.SemaphoreType.DMA,
            pltpu.SemaphoreType.DMA,
            pltpu.SemaphoreType.DMA,
            pltpu.SemaphoreType.DMA,
        ],
    )
    def k(m_hbm, i8_hbm, agg_hbm, ida, idb, mba, mbb, zbuf, acc,
          sia, sib, sma, smb):
        c = lax.axis_index("c")
        s = lax.axis_index("s")
        base = s * EPT

        @pl.loop(0, zrows)
        def _zrow(i):
            for j in range(F // 2 // 16):
                zbuf[i, pl.ds(j * 16, 16)] = jnp.zeros((16,), jnp.float32)

        @pl.loop(0, (n_zchunks + N_TILES - 1) // N_TILES)
        def _zcp(t):
            ch = t * N_TILES + s

            @pl.when(ch < n_zchunks)
            def _():
                pltpu.sync_copy(zbuf, acc.at[pl.ds(ch * zrows, zrows), :])

        plsc.subcore_barrier()

        def start(a, idxv, mbuf, si, sm):
            e0 = base + a * KS
            pltpu.async_copy(i8_hbm.at[pl.ds(4 * NE + e0, KS)], idxv, si)
            pltpu.async_copy(m_hbm.at[c, pl.ds(e0, KS), :], mbuf, sm)

        def finish(a, idxv, mbuf, si, sm):
            e0 = base + a * KS
            pltpu.make_async_copy(
                i8_hbm.at[pl.ds(4 * NE + e0, KS)], idxv, si).wait()
            pltpu.make_async_copy(
                m_hbm.at[c, pl.ds(e0, KS), :], mbuf, sm).wait()
            pltpu.sync_copy(mbuf, acc.at[idxv], add=True)

        start(0, ida, mba, sia, sma)

        @pl.loop(0, nch // 2)
        def _pair(p):
            a = 2 * p

            @pl.when(a + 1 < nch)
            def _():
                start(a + 1, idb, mbb, sib, smb)

            finish(a, ida, mba, sia, sma)

            @pl.when(a + 2 < nch)
            def _():
                start(a + 2, ida, mba, sia, sma)

            @pl.when(a + 1 < nch)
            def _():
                finish(a + 1, idb, mbb, sib, smb)

        if nch % 2 == 1:
            finish(nch - 1, ida, mba, sia, sma)

        plsc.subcore_barrier()

        @pl.loop(0, (n_zchunks + N_TILES - 1) // N_TILES)
        def _dump(t):
            ch = t * N_TILES + s

            @pl.when(ch < n_zchunks)
            def _():
                r0 = ch * zrows
                pltpu.sync_copy(acc.at[pl.ds(r0, zrows), :],
                                agg_hbm.at[c, pl.ds(r0, zrows), :])

    return k(m2, idx8)


def kernel(x, edge_index, edge_attr, batch,
           Wf1, bf1, Ws1, bs1, Wf2, bf2, Ws2, bs2, Wfc, bfc):
    ei = edge_index.astype(jnp.int32)
    idx8 = _build_idx(ei)

    w4_1 = jnp.concatenate(
        [Wf1[:F], Ws1[:F], Wf1[F:2 * F], Ws1[F:2 * F]], axis=1)
    w4_2 = jnp.concatenate(
        [Wf2[:F], Ws2[:F], Wf2[F:2 * F], Ws2[F:2 * F]], axis=1)
    we1 = jnp.concatenate([Wf1[2 * F:], Ws1[2 * F:]], axis=1)
    we2 = jnp.concatenate([Wf2[2 * F:], Ws2[2 * F:]], axis=1)
    b1 = jnp.broadcast_to(jnp.concatenate([bf1, bs1])[None, :], (8, 2 * F))
    b2 = jnp.broadcast_to(jnp.concatenate([bf2, bs2])[None, :], (8, 2 * F))

    p1 = _tables1(x, w4_1)
    idxf = idx8.reshape(8 * NE)
    g1 = _sc_gather(p1.reshape(4 * NN, F), idxf)
    m1 = _edge_mlp(g1, edge_attr, we1, b1)
    agg1 = _sc_scatter(m1, idxf)

    h1, p2 = _tables2(x, agg1, w4_2)
    g2 = _sc_gather(p2.reshape(4 * NN, F), idxf)
    m2 = _edge_mlp(g2, edge_attr, we2, b2)
    agg2 = _sc_scatter(m2, idxf)

    batch3 = batch.astype(jnp.int32).reshape(NN // NODE_BLK, 1, NODE_BLK)
    return _final(h1, agg2, batch3, Wfc, jnp.reshape(bfc, (1, 1)))


# speedup vs baseline: 2.5676x; 1.1839x over previous
import functools

import jax
import jax.numpy as jnp
from jax import lax
from jax.experimental import pallas as pl
from jax.experimental.pallas import tpu as pltpu
from jax.experimental.pallas import tpu_sc as plsc

NN = 10000
NE = 160000
F = 256
DE = 16
NG = 64

NODE_BLK = 1000
EDGE_BLK = 1000
N_TILES = 16
KG = 80
KS = 80
EPT = NE // N_TILES


def _idx_body(ei_ref, out_ref):
    s = ei_ref[0:1, :]
    d = ei_ref[1:2, :]
    out_ref[0:1, :] = d * 4
    out_ref[1:2, :] = d * 4 + 1
    out_ref[2:3, :] = s * 4 + 2
    out_ref[3:4, :] = s * 4 + 3
    out_ref[4:5, :] = d
    out_ref[5:6, :] = d
    out_ref[6:7, :] = d
    out_ref[7:8, :] = d


def _build_idx(ei):
    return pl.pallas_call(
        _idx_body,
        out_shape=jax.ShapeDtypeStruct((8, NE), jnp.int32),
    )(ei)


def _pack256(q):
    qb = q.astype(jnp.bfloat16)
    lo = lax.bitcast_convert_type(qb[:, :128], jnp.uint16).astype(jnp.uint32)
    hi = lax.bitcast_convert_type(qb[:, 128:], jnp.uint16).astype(jnp.uint32)
    return lax.bitcast_convert_type(lo | (hi << 16), jnp.int32)


def _pack_tables(p):
    return jnp.concatenate(
        [_pack256(p[:, k * F:(k + 1) * F]) for k in range(4)], axis=1)


def _tab1_body(x_ref, w_ref, p_ref):
    p = jnp.dot(x_ref[...], w_ref[...], preferred_element_type=jnp.float32)
    p_ref[...] = _pack_tables(p)


def _tables1(x, w4):
    return pl.pallas_call(
        _tab1_body,
        grid=(NN // NODE_BLK,),
        in_specs=[
            pl.BlockSpec((NODE_BLK, F), lambda i: (i, 0)),
            pl.BlockSpec((F, 4 * F), lambda i: (0, 0)),
        ],
        out_specs=pl.BlockSpec((NODE_BLK, 2 * F), lambda i: (i, 0)),
        out_shape=jax.ShapeDtypeStruct((NN, 2 * F), jnp.int32),
    )(x, w4)


def _tab2_body(x_ref, agg_ref, w_ref, h_ref, p_ref):
    h = x_ref[...] + jnp.concatenate([agg_ref[0], agg_ref[1]], axis=-1)
    h_ref[...] = h
    p = jnp.dot(h, w_ref[...], preferred_element_type=jnp.float32)
    p_ref[...] = _pack_tables(p)


def _tables2(x, agg, w4):
    return pl.pallas_call(
        _tab2_body,
        grid=(NN // NODE_BLK,),
        in_specs=[
            pl.BlockSpec((NODE_BLK, F), lambda i: (i, 0)),
            pl.BlockSpec((2, NODE_BLK, F // 2), lambda i: (0, i, 0)),
            pl.BlockSpec((F, 4 * F), lambda i: (0, 0)),
        ],
        out_specs=[
            pl.BlockSpec((NODE_BLK, F), lambda i: (i, 0)),
            pl.BlockSpec((NODE_BLK, 2 * F), lambda i: (i, 0)),
        ],
        out_shape=[
            jax.ShapeDtypeStruct((NN, F), jnp.float32),
            jax.ShapeDtypeStruct((NN, 2 * F), jnp.int32),
        ],
    )(x, agg, w4)


def _unpk(gi):
    lo = lax.bitcast_convert_type(jnp.left_shift(gi, 16), jnp.float32)
    hi = lax.bitcast_convert_type(gi & jnp.int32(-65536), jnp.float32)
    return lo, hi


def _softplus(z):
    return jnp.maximum(z, 0.0) + jnp.log1p(jnp.exp(-jnp.abs(z)))


def _edge_body(gd_ref, gs_ref, ea_ref, we_ref, b_ref, m_ref):
    h = F // 2
    e = jnp.dot(ea_ref[...], we_ref[...], preferred_element_type=jnp.float32)
    fd_lo, fd_hi = _unpk(gd_ref[0])
    sd_lo, sd_hi = _unpk(gd_ref[1])
    fs_lo, fs_hi = _unpk(gs_ref[0])
    ss_lo, ss_hi = _unpk(gs_ref[1])
    zf_lo = fd_lo + fs_lo + e[:, 0:h] + b_ref[0:1, 0:h]
    zf_hi = fd_hi + fs_hi + e[:, h:F] + b_ref[0:1, h:F]
    zs_lo = sd_lo + ss_lo + e[:, F:F + h] + b_ref[0:1, F:F + h]
    zs_hi = sd_hi + ss_hi + e[:, F + h:] + b_ref[0:1, F + h:]
    m_ref[0] = jax.nn.sigmoid(zf_lo) * _softplus(zs_lo)
    m_ref[1] = jax.nn.sigmoid(zf_hi) * _softplus(zs_hi)


def _edge_mlp(gd, gs, ea, we, b):
    return pl.pallas_call(
        _edge_body,
        grid=(NE // EDGE_BLK,),
        in_specs=[
            pl.BlockSpec((2, EDGE_BLK, F // 2), lambda i: (0, i, 0)),
            pl.BlockSpec((2, EDGE_BLK, F // 2), lambda i: (0, i, 0)),
            pl.BlockSpec((EDGE_BLK, DE), lambda i: (i, 0)),
            pl.BlockSpec((DE, 2 * F), lambda i: (0, 0)),
            pl.BlockSpec((8, 2 * F), lambda i: (0, 0)),
        ],
        out_specs=pl.BlockSpec((2, EDGE_BLK, F // 2), lambda i: (0, i, 0)),
        out_shape=jax.ShapeDtypeStruct((2, NE, F // 2), jnp.float32),
    )(gd, gs, ea, we, b)


def _final_body(h_ref, agg_ref, batch_ref, wfc_ref, bfc_ref, out_ref,
                pooled, cnt):
    i = pl.program_id(0)

    @pl.when(i == 0)
    def _():
        pooled[...] = jnp.zeros_like(pooled)
        cnt[...] = jnp.zeros_like(cnt)

    h2 = h_ref[...] + jnp.concatenate([agg_ref[0], agg_ref[1]], axis=-1)
    b = batch_ref[0, 0, :]
    ids = lax.broadcasted_iota(jnp.int32, (NG, NODE_BLK), 0)
    mask_t = (ids == b[None, :]).astype(jnp.float32)
    pooled[...] += jnp.dot(mask_t, h2, preferred_element_type=jnp.float32)
    cnt[...] += jnp.broadcast_to(
        jnp.sum(mask_t, axis=1, keepdims=True), (NG, 128))

    @pl.when(i == pl.num_programs(0) - 1)
    def _():
        out_ref[...] = (jnp.dot(pooled[...], wfc_ref[...],
                                preferred_element_type=jnp.float32)
                        + cnt[:, 0:1] * bfc_ref[0, 0])


def _final(h1, agg2, batch3, wfc, bfc11):
    return pl.pallas_call(
        _final_body,
        grid=(NN // NODE_BLK,),
        in_specs=[
            pl.BlockSpec((NODE_BLK, F), lambda i: (i, 0)),
            pl.BlockSpec((2, NODE_BLK, F // 2), lambda i: (0, i, 0)),
            pl.BlockSpec((1, 1, NODE_BLK), lambda i: (i, 0, 0)),
            pl.BlockSpec((F, 1), lambda i: (0, 0)),
            pl.BlockSpec((1, 1), lambda i: (0, 0),
                         memory_space=pltpu.SMEM),
        ],
        out_specs=pl.BlockSpec((NG, 1), lambda i: (0, 0)),
        out_shape=jax.ShapeDtypeStruct((NG, 1), jnp.float32),
        scratch_shapes=[
            pltpu.VMEM((NG, F), jnp.float32),
            pltpu.VMEM((NG, 128), jnp.float32),
        ],
    )(h1, agg2, batch3, wfc, bfc11)


def _mesh():
    return plsc.VectorSubcoreMesh(core_axis_name="c", subcore_axis_name="s",
                                  num_cores=2, num_subcores=N_TILES)


def _sc_gather(p40, idx8):
    nch = EPT // KG
    fp = F // 2

    @functools.partial(
        pl.kernel,
        out_type=[
            jax.ShapeDtypeStruct((2, NE, fp), jnp.int32),
            jax.ShapeDtypeStruct((2, NE, fp), jnp.int32),
        ],
        mesh=_mesh(),
        scratch_types=[
            pltpu.VMEM((EPT,), jnp.int32),
            pltpu.VMEM((EPT,), jnp.int32),
            pltpu.VMEM((KG, fp), jnp.int32),
            pltpu.VMEM((KG, fp), jnp.int32),
            pltpu.VMEM((KG, fp), jnp.int32),
            pltpu.VMEM((KG, fp), jnp.int32),
            pltpu.SemaphoreType.DMA,
            pltpu.SemaphoreType.DMA,
            pltpu.SemaphoreType.DMA,
            pltpu.SemaphoreType.DMA,
            pltpu.SemaphoreType.DMA,
            pltpu.SemaphoreType.DMA,
            pltpu.SemaphoreType.DMA,
            pltpu.SemaphoreType.DMA,
        ],
    )
    def k(p_hbm, i8_hbm, gd_hbm, gs_hbm, idxd, idxs,
          bda, bsa, bdb, bsb, sda, ssa, sdb, ssb, wda, wsa, wdb, wsb):
        c = lax.axis_index("c")
        s = lax.axis_index("s")
        base = s * EPT
        pltpu.sync_copy(i8_hbm.at[pl.ds(c * NE + base, EPT)], idxd)
        pltpu.sync_copy(i8_hbm.at[pl.ds((2 + c) * NE + base, EPT)], idxs)

        def start(a, bufd, bufs, semd, sems, wd, ws, first):
            sl = pl.ds(a * KG, KG)
            osl = pl.ds(base + a * KG, KG)
            if not first:
                pltpu.make_async_copy(bufd, gd_hbm.at[c, osl, :], wd).wait()
                pltpu.make_async_copy(bufs, gs_hbm.at[c, osl, :], ws).wait()
            pltpu.async_copy(p_hbm.at[idxd.at[sl]], bufd, semd)
            pltpu.async_copy(p_hbm.at[idxs.at[sl]], bufs, sems)

        def finish(a, bufd, bufs, semd, sems, wd, ws):
            sl = pl.ds(a * KG, KG)
            osl = pl.ds(base + a * KG, KG)
            pltpu.make_async_copy(p_hbm.at[idxd.at[sl]], bufd, semd).wait()
            pltpu.make_async_copy(p_hbm.at[idxs.at[sl]], bufs, sems).wait()
            pltpu.async_copy(bufd, gd_hbm.at[c, osl, :], wd)
            pltpu.async_copy(bufs, gs_hbm.at[c, osl, :], ws)

        start(0, bda, bsa, sda, ssa, wda, wsa, True)
        start(1, bdb, bsb, sdb, ssb, wdb, wsb, True)

        @pl.loop(0, nch // 2)
        def _pair(p):
            a = 2 * p
            finish(a, bda, bsa, sda, ssa, wda, wsa)

            @pl.when(a + 2 < nch)
            def _():
                start(a + 2, bda, bsa, sda, ssa, wda, wsa, False)

            finish(a + 1, bdb, bsb, sdb, ssb, wdb, wsb)

            @pl.when(a + 3 < nch)
            def _():
                start(a + 3, bdb, bsb, sdb, ssb, wdb, wsb, False)

        finish(nch - 1, bda, bsa, sda, ssa, wda, wsa)
        osl_last = pl.ds(base + (nch - 1) * KG, KG)
        osl_prev = pl.ds(base + (nch - 2) * KG, KG)
        pltpu.make_async_copy(bda, gd_hbm.at[c, osl_last, :], wda).wait()
        pltpu.make_async_copy(bsa, gs_hbm.at[c, osl_last, :], wsa).wait()
        pltpu.make_async_copy(bdb, gd_hbm.at[c, osl_prev, :], wdb).wait()
        pltpu.make_async_copy(bsb, gs_hbm.at[c, osl_prev, :], wsb).wait()

    return k(p40, idx8)


def _sc_scatter(m2, idx8):
    nch = EPT // KS
    zrows = 80
    n_zchunks = NN // zrows

    @functools.partial(
        pl.kernel,
        out_type=jax.ShapeDtypeStruct((2, NN, F // 2), jnp.float32),
        mesh=_mesh(),
        scratch_types=[
            pltpu.VMEM((KS,), jnp.int32),
            pltpu.VMEM((KS,), jnp.int32),
            pltpu.VMEM((KS, F // 2), jnp.float32),
            pltpu.VMEM((KS, F // 2), jnp.float32),
            pltpu.VMEM((zrows, F // 2), jnp.float32),
            pltpu.VMEM_SHARED((NN, F // 2), jnp.float32),
            pltpu.SemaphoreType.DMA,
            pltpu.SemaphoreType.DMA,
            pltpu.SemaphoreType.DMA,
            pltpu.SemaphoreType.DMA,
        ],
    )
    def k(m_hbm, i8_hbm, agg_hbm, ida, idb, mba, mbb, zbuf, acc,
          sia, sib, sma, smb):
        c = lax.axis_index("c")
        s = lax.axis_index("s")
        base = s * EPT

        @pl.loop(0, zrows)
        def _zrow(i):
            for j in range(F // 2 // 16):
                zbuf[i, pl.ds(j * 16, 16)] = jnp.zeros((16,), jnp.float32)

        @pl.loop(0, (n_zchunks + N_TILES - 1) // N_TILES)
        def _zcp(t):
            ch = t * N_TILES + s

            @pl.when(ch < n_zchunks)
            def _():
                pltpu.sync_copy(zbuf, acc.at[pl.ds(ch * zrows, zrows), :])

        plsc.subcore_barrier()

        def start(a, idxv, mbuf, si, sm):
            e0 = base + a * KS
            pltpu.async_copy(i8_hbm.at[pl.ds(4 * NE + e0, KS)], idxv, si)
            pltpu.async_copy(m_hbm.at[c, pl.ds(e0, KS), :], mbuf, sm)

        def finish(a, idxv, mbuf, si, sm):
            e0 = base + a * KS
            pltpu.make_async_copy(
                i8_hbm.at[pl.ds(4 * NE + e0, KS)], idxv, si).wait()
            pltpu.make_async_copy(
                m_hbm.at[c, pl.ds(e0, KS), :], mbuf, sm).wait()
            pltpu.sync_copy(mbuf, acc.at[idxv], add=True)

        start(0, ida, mba, sia, sma)

        @pl.loop(0, nch // 2)
        def _pair(p):
            a = 2 * p

            @pl.when(a + 1 < nch)
            def _():
                start(a + 1, idb, mbb, sib, smb)

            finish(a, ida, mba, sia, sma)

            @pl.when(a + 2 < nch)
            def _():
                start(a + 2, ida, mba, sia, sma)

            @pl.when(a + 1 < nch)
            def _():
                finish(a + 1, idb, mbb, sib, smb)

        if nch % 2 == 1:
            finish(nch - 1, ida, mba, sia, sma)

        plsc.subcore_barrier()

        @pl.loop(0, (n_zchunks + N_TILES - 1) // N_TILES)
        def _dump(t):
            ch = t * N_TILES + s

            @pl.when(ch < n_zchunks)
            def _():
                r0 = ch * zrows
                pltpu.sync_copy(acc.at[pl.ds(r0, zrows), :],
                                agg_hbm.at[c, pl.ds(r0, zrows), :])

    return k(m2, idx8)


def kernel(x, edge_index, edge_attr, batch,
           Wf1, bf1, Ws1, bs1, Wf2, bf2, Ws2, bs2, Wfc, bfc):
    ei = edge_index.astype(jnp.int32)
    idx8 = _build_idx(ei)

    w4_1 = jnp.concatenate(
        [Wf1[:F], Ws1[:F], Wf1[F:2 * F], Ws1[F:2 * F]], axis=1)
    w4_2 = jnp.concatenate(
        [Wf2[:F], Ws2[:F], Wf2[F:2 * F], Ws2[F:2 * F]], axis=1)
    we1 = jnp.concatenate([Wf1[2 * F:], Ws1[2 * F:]], axis=1)
    we2 = jnp.concatenate([Wf2[2 * F:], Ws2[2 * F:]], axis=1)
    b1 = jnp.broadcast_to(jnp.concatenate([bf1, bs1])[None, :], (8, 2 * F))
    b2 = jnp.broadcast_to(jnp.concatenate([bf2, bs2])[None, :], (8, 2 * F))

    p1 = _tables1(x, w4_1)
    idxf = idx8.reshape(8 * NE)
    gd1, gs1 = _sc_gather(p1.reshape(4 * NN, F // 2), idxf)
    m1 = _edge_mlp(gd1, gs1, edge_attr, we1, b1)
    agg1 = _sc_scatter(m1, idxf)

    h1, p2 = _tables2(x, agg1, w4_2)
    gd2, gs2 = _sc_gather(p2.reshape(4 * NN, F // 2), idxf)
    m2 = _edge_mlp(gd2, gs2, edge_attr, we2, b2)
    agg2 = _sc_scatter(m2, idxf)

    batch3 = batch.astype(jnp.int32).reshape(NN // NODE_BLK, 1, NODE_BLK)
    return _final(h1, agg2, batch3, Wfc, jnp.reshape(bfc, (1, 1)))


# speedup vs baseline: 2.7656x; 1.0771x over previous
import functools

import jax
import jax.numpy as jnp
from jax import lax
from jax.experimental import pallas as pl
from jax.experimental.pallas import tpu as pltpu
from jax.experimental.pallas import tpu_sc as plsc

NN = 10000
NE = 160000
F = 256
DE = 16
NG = 64

NODE_BLK = 1000
EDGE_BLK = 1000
N_TILES = 16
NSLAB = 2
NES = NE // NSLAB
KG = 40
KS = 40
EPT = NES // N_TILES


def _idx_body(ei_ref, out_ref):
    s = ei_ref[0:1, :]
    d = ei_ref[1:2, :]
    out_ref[0:1, :] = d * 4
    out_ref[1:2, :] = d * 4 + 1
    out_ref[2:3, :] = s * 4 + 2
    out_ref[3:4, :] = s * 4 + 3
    out_ref[4:5, :] = d
    out_ref[5:6, :] = d
    out_ref[6:7, :] = d
    out_ref[7:8, :] = d


def _build_idx(ei):
    return pl.pallas_call(
        _idx_body,
        out_shape=jax.ShapeDtypeStruct((8, NE), jnp.int32),
    )(ei)


def _pack256(q):
    qb = q.astype(jnp.bfloat16)
    lo = lax.bitcast_convert_type(qb[:, :128], jnp.uint16).astype(jnp.uint32)
    hi = lax.bitcast_convert_type(qb[:, 128:], jnp.uint16).astype(jnp.uint32)
    return lax.bitcast_convert_type(lo | (hi << 16), jnp.int32)


def _pack_tables(p):
    return jnp.concatenate(
        [_pack256(p[:, k * F:(k + 1) * F]) for k in range(4)], axis=1)


def _tab1_body(x_ref, w_ref, p_ref):
    p = jnp.dot(x_ref[...], w_ref[...], preferred_element_type=jnp.float32)
    p_ref[...] = _pack_tables(p)


def _tables1(x, w4):
    return pl.pallas_call(
        _tab1_body,
        grid=(NN // NODE_BLK,),
        in_specs=[
            pl.BlockSpec((NODE_BLK, F), lambda i: (i, 0)),
            pl.BlockSpec((F, 4 * F), lambda i: (0, 0)),
        ],
        out_specs=pl.BlockSpec((NODE_BLK, 2 * F), lambda i: (i, 0)),
        out_shape=jax.ShapeDtypeStruct((NN, 2 * F), jnp.int32),
    )(x, w4)


def _tab2_body(x_ref, agga_ref, aggb_ref, w_ref, h_ref, p_ref):
    agg0 = agga_ref[0] + aggb_ref[0]
    agg1 = agga_ref[1] + aggb_ref[1]
    h = x_ref[...] + jnp.concatenate([agg0, agg1], axis=-1)
    h_ref[...] = h
    p = jnp.dot(h, w_ref[...], preferred_element_type=jnp.float32)
    p_ref[...] = _pack_tables(p)


def _tables2(x, agga, aggb, w4):
    return pl.pallas_call(
        _tab2_body,
        grid=(NN // NODE_BLK,),
        in_specs=[
            pl.BlockSpec((NODE_BLK, F), lambda i: (i, 0)),
            pl.BlockSpec((2, NODE_BLK, F // 2), lambda i: (0, i, 0)),
            pl.BlockSpec((2, NODE_BLK, F // 2), lambda i: (0, i, 0)),
            pl.BlockSpec((F, 4 * F), lambda i: (0, 0)),
        ],
        out_specs=[
            pl.BlockSpec((NODE_BLK, F), lambda i: (i, 0)),
            pl.BlockSpec((NODE_BLK, 2 * F), lambda i: (i, 0)),
        ],
        out_shape=[
            jax.ShapeDtypeStruct((NN, F), jnp.float32),
            jax.ShapeDtypeStruct((NN, 2 * F), jnp.int32),
        ],
    )(x, agga, aggb, w4)


def _unpk(gi):
    lo = lax.bitcast_convert_type(jnp.left_shift(gi, 16), jnp.float32)
    hi = lax.bitcast_convert_type(gi & jnp.int32(-65536), jnp.float32)
    return lo, hi


def _softplus(z):
    return jnp.maximum(z, 0.0) + jnp.log1p(jnp.exp(-jnp.abs(z)))


def _edge_body(gd_ref, gs_ref, ea_ref, we_ref, b_ref, m_ref):
    h = F // 2
    e = jnp.dot(ea_ref[...], we_ref[...], preferred_element_type=jnp.float32)
    fd_lo, fd_hi = _unpk(gd_ref[0])
    sd_lo, sd_hi = _unpk(gd_ref[1])
    fs_lo, fs_hi = _unpk(gs_ref[0])
    ss_lo, ss_hi = _unpk(gs_ref[1])
    zf_lo = fd_lo + fs_lo + e[:, 0:h] + b_ref[0:1, 0:h]
    zf_hi = fd_hi + fs_hi + e[:, h:F] + b_ref[0:1, h:F]
    zs_lo = sd_lo + ss_lo + e[:, F:F + h] + b_ref[0:1, F:F + h]
    zs_hi = sd_hi + ss_hi + e[:, F + h:] + b_ref[0:1, F + h:]
    m_ref[0] = jax.nn.sigmoid(zf_lo) * _softplus(zs_lo)
    m_ref[1] = jax.nn.sigmoid(zf_hi) * _softplus(zs_hi)


def _edge_mlp(gd, gs, ea, we, b, slab):
    off = slab * (NES // EDGE_BLK)
    return pl.pallas_call(
        _edge_body,
        grid=(NES // EDGE_BLK,),
        in_specs=[
            pl.BlockSpec((2, EDGE_BLK, F // 2), lambda i: (0, i, 0)),
            pl.BlockSpec((2, EDGE_BLK, F // 2), lambda i: (0, i, 0)),
            pl.BlockSpec((EDGE_BLK, DE), lambda i: (i + off, 0)),
            pl.BlockSpec((DE, 2 * F), lambda i: (0, 0)),
            pl.BlockSpec((8, 2 * F), lambda i: (0, 0)),
        ],
        out_specs=pl.BlockSpec((2, EDGE_BLK, F // 2), lambda i: (0, i, 0)),
        out_shape=jax.ShapeDtypeStruct((2, NES, F // 2), jnp.float32),
    )(gd, gs, ea, we, b)


def _final_body(h_ref, agga_ref, aggb_ref, batch_ref, wfc_ref, bfc_ref,
                out_ref, pooled, cnt):
    i = pl.program_id(0)

    @pl.when(i == 0)
    def _():
        pooled[...] = jnp.zeros_like(pooled)
        cnt[...] = jnp.zeros_like(cnt)

    h2 = h_ref[...] + jnp.concatenate(
        [agga_ref[0] + aggb_ref[0], agga_ref[1] + aggb_ref[1]], axis=-1)
    b = batch_ref[0, 0, :]
    ids = lax.broadcasted_iota(jnp.int32, (NG, NODE_BLK), 0)
    mask_t = (ids == b[None, :]).astype(jnp.float32)
    pooled[...] += jnp.dot(mask_t, h2, preferred_element_type=jnp.float32)
    cnt[...] += jnp.broadcast_to(
        jnp.sum(mask_t, axis=1, keepdims=True), (NG, 128))

    @pl.when(i == pl.num_programs(0) - 1)
    def _():
        out_ref[...] = (jnp.dot(pooled[...], wfc_ref[...],
                                preferred_element_type=jnp.float32)
                        + cnt[:, 0:1] * bfc_ref[0, 0])


def _final(h1, agg2a, agg2b, batch3, wfc, bfc11):
    return pl.pallas_call(
        _final_body,
        grid=(NN // NODE_BLK,),
        in_specs=[
            pl.BlockSpec((NODE_BLK, F), lambda i: (i, 0)),
            pl.BlockSpec((2, NODE_BLK, F // 2), lambda i: (0, i, 0)),
            pl.BlockSpec((2, NODE_BLK, F // 2), lambda i: (0, i, 0)),
            pl.BlockSpec((1, 1, NODE_BLK), lambda i: (i, 0, 0)),
            pl.BlockSpec((F, 1), lambda i: (0, 0)),
            pl.BlockSpec((1, 1), lambda i: (0, 0),
                         memory_space=pltpu.SMEM),
        ],
        out_specs=pl.BlockSpec((NG, 1), lambda i: (0, 0)),
        out_shape=jax.ShapeDtypeStruct((NG, 1), jnp.float32),
        scratch_shapes=[
            pltpu.VMEM((NG, F), jnp.float32),
            pltpu.VMEM((NG, 128), jnp.float32),
        ],
    )(h1, agg2a, agg2b, batch3, wfc, bfc11)


def _mesh():
    return plsc.VectorSubcoreMesh(core_axis_name="c", subcore_axis_name="s",
                                  num_cores=2, num_subcores=N_TILES)


def _sc_gather(p40, idx8):
    nch = EPT // KG
    fp = F // 2

    @functools.partial(
        pl.kernel,
        out_type=[
            jax.ShapeDtypeStruct((2, NES, fp), jnp.int32),
            jax.ShapeDtypeStruct((2, NES, fp), jnp.int32),
        ],
        mesh=_mesh(),
        scratch_types=[
            pltpu.VMEM((EPT,), jnp.int32),
            pltpu.VMEM((EPT,), jnp.int32),
            pltpu.VMEM((KG, fp), jnp.int32),
            pltpu.VMEM((KG, fp), jnp.int32),
            pltpu.VMEM((KG, fp), jnp.int32),
            pltpu.VMEM((KG, fp), jnp.int32),
            pltpu.SemaphoreType.DMA,
            pltpu.SemaphoreType.DMA,
            pltpu.SemaphoreType.DMA,
            pltpu.SemaphoreType.DMA,
            pltpu.SemaphoreType.DMA,
            pltpu.SemaphoreType.DMA,
            pltpu.SemaphoreType.DMA,
            pltpu.SemaphoreType.DMA,
        ],
    )
    def k(p_hbm, i8_hbm, gd_hbm, gs_hbm, idxd, idxs,
          bda, bsa, bdb, bsb, sda, ssa, sdb, ssb, wda, wsa, wdb, wsb):
        c = lax.axis_index("c")
        s = lax.axis_index("s")
        base = s * EPT
        pltpu.sync_copy(i8_hbm.at[pl.ds(c * NES + base, EPT)], idxd)
        pltpu.sync_copy(i8_hbm.at[pl.ds((2 + c) * NES + base, EPT)], idxs)

        def start(a, bufd, bufs, semd, sems, wd, ws, first):
            sl = pl.ds(a * KG, KG)
            osl = pl.ds(base + a * KG, KG)
            if not first:
                pltpu.make_async_copy(bufd, gd_hbm.at[c, osl, :], wd).wait()
                pltpu.make_async_copy(bufs, gs_hbm.at[c, osl, :], ws).wait()
            pltpu.async_copy(p_hbm.at[idxd.at[sl]], bufd, semd)
            pltpu.async_copy(p_hbm.at[idxs.at[sl]], bufs, sems)

        def finish(a, bufd, bufs, semd, sems, wd, ws):
            sl = pl.ds(a * KG, KG)
            osl = pl.ds(base + a * KG, KG)
            pltpu.make_async_copy(p_hbm.at[idxd.at[sl]], bufd, semd).wait()
            pltpu.make_async_copy(p_hbm.at[idxs.at[sl]], bufs, sems).wait()
            pltpu.async_copy(bufd, gd_hbm.at[c, osl, :], wd)
            pltpu.async_copy(bufs, gs_hbm.at[c, osl, :], ws)

        start(0, bda, bsa, sda, ssa, wda, wsa, True)
        start(1, bdb, bsb, sdb, ssb, wdb, wsb, True)

        @pl.loop(0, nch // 2)
        def _pair(p):
            a = 2 * p
            finish(a, bda, bsa, sda, ssa, wda, wsa)

            @pl.when(a + 2 < nch)
            def _():
                start(a + 2, bda, bsa, sda, ssa, wda, wsa, False)

            finish(a + 1, bdb, bsb, sdb, ssb, wdb, wsb)

            @pl.when(a + 3 < nch)
            def _():
                start(a + 3, bdb, bsb, sdb, ssb, wdb, wsb, False)

        finish(nch - 1, bda, bsa, sda, ssa, wda, wsa)
        osl_last = pl.ds(base + (nch - 1) * KG, KG)
        osl_prev = pl.ds(base + (nch - 2) * KG, KG)
        pltpu.make_async_copy(bda, gd_hbm.at[c, osl_last, :], wda).wait()
        pltpu.make_async_copy(bsa, gs_hbm.at[c, osl_last, :], wsa).wait()
        pltpu.make_async_copy(bdb, gd_hbm.at[c, osl_prev, :], wdb).wait()
        pltpu.make_async_copy(bsb, gs_hbm.at[c, osl_prev, :], wsb).wait()

    return k(p40, idx8)


def _sc_scatter(m2, idx8):
    nch = EPT // KS
    zrows = 80
    n_zchunks = NN // zrows

    @functools.partial(
        pl.kernel,
        out_type=jax.ShapeDtypeStruct((2, NN, F // 2), jnp.float32),
        mesh=_mesh(),
        scratch_types=[
            pltpu.VMEM((KS,), jnp.int32),
            pltpu.VMEM((KS,), jnp.int32),
            pltpu.VMEM((KS, F // 2), jnp.float32),
            pltpu.VMEM((KS, F // 2), jnp.float32),
            pltpu.VMEM((zrows, F // 2), jnp.float32),
            pltpu.VMEM_SHARED((NN, F // 2), jnp.float32),
            pltpu.SemaphoreType.DMA,
            pltpu.SemaphoreType.DMA,
            pltpu.SemaphoreType.DMA,
            pltpu.SemaphoreType.DMA,
        ],
    )
    def k(m_hbm, i8_hbm, agg_hbm, ida, idb, mba, mbb, zbuf, acc,
          sia, sib, sma, smb):
        c = lax.axis_index("c")
        s = lax.axis_index("s")
        base = s * EPT

        @pl.loop(0, zrows)
        def _zrow(i):
            for j in range(F // 2 // 16):
                zbuf[i, pl.ds(j * 16, 16)] = jnp.zeros((16,), jnp.float32)

        @pl.loop(0, (n_zchunks + N_TILES - 1) // N_TILES)
        def _zcp(t):
            ch = t * N_TILES + s

            @pl.when(ch < n_zchunks)
            def _():
                pltpu.sync_copy(zbuf, acc.at[pl.ds(ch * zrows, zrows), :])

        plsc.subcore_barrier()

        def start(a, idxv, mbuf, si, sm):
            e0 = base + a * KS
            pltpu.async_copy(i8_hbm.at[pl.ds(4 * NES + e0, KS)], idxv, si)
            pltpu.async_copy(m_hbm.at[c, pl.ds(e0, KS), :], mbuf, sm)

        def finish(a, idxv, mbuf, si, sm):
            e0 = base + a * KS
            pltpu.make_async_copy(
                i8_hbm.at[pl.ds(4 * NES + e0, KS)], idxv, si).wait()
            pltpu.make_async_copy(
                m_hbm.at[c, pl.ds(e0, KS), :], mbuf, sm).wait()
            pltpu.sync_copy(mbuf, acc.at[idxv], add=True)

        start(0, ida, mba, sia, sma)

        @pl.loop(0, nch // 2)
        def _pair(p):
            a = 2 * p

            @pl.when(a + 1 < nch)
            def _():
                start(a + 1, idb, mbb, sib, smb)

            finish(a, ida, mba, sia, sma)

            @pl.when(a + 2 < nch)
            def _():
                start(a + 2, ida, mba, sia, sma)

            @pl.when(a + 1 < nch)
            def _():
                finish(a + 1, idb, mbb, sib, smb)

        if nch % 2 == 1:
            finish(nch - 1, ida, mba, sia, sma)

        plsc.subcore_barrier()

        @pl.loop(0, (n_zchunks + N_TILES - 1) // N_TILES)
        def _dump(t):
            ch = t * N_TILES + s

            @pl.when(ch < n_zchunks)
            def _():
                r0 = ch * zrows
                pltpu.sync_copy(acc.at[pl.ds(r0, zrows), :],
                                agg_hbm.at[c, pl.ds(r0, zrows), :])

    return k(m2, idx8)


def kernel(x, edge_index, edge_attr, batch,
           Wf1, bf1, Ws1, bs1, Wf2, bf2, Ws2, bs2, Wfc, bfc):
    ei = edge_index.astype(jnp.int32)
    idx8 = _build_idx(ei)

    w4_1 = jnp.concatenate(
        [Wf1[:F], Ws1[:F], Wf1[F:2 * F], Ws1[F:2 * F]], axis=1)
    w4_2 = jnp.concatenate(
        [Wf2[:F], Ws2[:F], Wf2[F:2 * F], Ws2[F:2 * F]], axis=1)
    we1 = jnp.concatenate([Wf1[2 * F:], Ws1[2 * F:]], axis=1)
    we2 = jnp.concatenate([Wf2[2 * F:], Ws2[2 * F:]], axis=1)
    b1 = jnp.broadcast_to(jnp.concatenate([bf1, bs1])[None, :], (8, 2 * F))
    b2 = jnp.broadcast_to(jnp.concatenate([bf2, bs2])[None, :], (8, 2 * F))

    idxh = [idx8[:, s * NES:(s + 1) * NES].reshape(8 * NES)
            for s in range(NSLAB)]

    def layer(ptab, we, b):
        aggs = []
        for s in range(NSLAB):
            gd, gs = _sc_gather(ptab, idxh[s])
            m = _edge_mlp(gd, gs, edge_attr, we, b, s)
            aggs.append(_sc_scatter(m, idxh[s]))
        return aggs

    p1 = _tables1(x, w4_1)
    agg1a, agg1b = layer(p1.reshape(4 * NN, F // 2), we1, b1)
    h1, p2 = _tables2(x, agg1a, agg1b, w4_2)
    agg2a, agg2b = layer(p2.reshape(4 * NN, F // 2), we2, b2)

    batch3 = batch.astype(jnp.int32).reshape(NN // NODE_BLK, 1, NODE_BLK)
    return _final(h1, agg2a, agg2b, batch3, Wfc, jnp.reshape(bfc, (1, 1)))
